# Initial kernel scaffold; baseline (speedup 1.0000x reference)
import jax
import jax.numpy as jnp
from jax.experimental import pallas as pl


def kernel(prefix, table):
    raise NotImplementedError("write your pallas kernel here")


# speedup vs baseline: 3.4017x; 3.4017x over previous
import functools

import jax
import jax.numpy as jnp
from jax import lax
from jax.experimental import pallas as pl
from jax.experimental.pallas import tpu as pltpu
from jax.experimental.pallas import tpu_sc as plsc

BATCH = 16384
SEQ = 200
HIDDEN = 128

NUM_CORES = 2
NUM_SUBCORES = 16
NUM_WORKERS = NUM_CORES * NUM_SUBCORES

TOTAL = BATCH * SEQ
PER_WORKER = TOTAL // NUM_WORKERS
CHUNK = 128
STEPS = PER_WORKER // CHUNK


def _sc_gather(idx_hbm, table_hbm, out_hbm, idx_v, rows_v, sem):
    wid = lax.axis_index("s") * NUM_CORES + lax.axis_index("c")
    base = wid * PER_WORKER

    def step(g, _):
        off = base + g * CHUNK
        pltpu.sync_copy(idx_hbm.at[pl.ds(off, CHUNK)], idx_v)
        pltpu.async_copy(table_hbm.at[idx_v], rows_v, sem).wait()
        pltpu.sync_copy(rows_v, out_hbm.at[pl.ds(off, CHUNK)])
        return ()

    lax.fori_loop(0, STEPS, step, ())


@jax.jit
def kernel(prefix, table):
    idx = prefix.reshape(TOTAL).astype(jnp.int32)
    mesh = plsc.VectorSubcoreMesh(core_axis_name="c", subcore_axis_name="s")
    run = pl.kernel(
        _sc_gather,
        out_type=jax.ShapeDtypeStruct((TOTAL, HIDDEN), jnp.float32),
        mesh=mesh,
        scratch_types=[
            pltpu.VMEM((CHUNK,), jnp.int32),
            pltpu.VMEM((CHUNK, HIDDEN), jnp.float32),
            pltpu.SemaphoreType.DMA,
        ],
    )
    out = run(idx, table)
    return out.reshape(BATCH, SEQ, HIDDEN)


# speedup vs baseline: 3.4143x; 1.0037x over previous
import jax
import jax.numpy as jnp
from jax import lax
from jax.experimental import pallas as pl
from jax.experimental.pallas import tpu as pltpu
from jax.experimental.pallas import tpu_sc as plsc

BATCH = 16384
SEQ = 200
HIDDEN = 128

NUM_CORES = 2
NUM_SUBCORES = 16
NUM_WORKERS = NUM_CORES * NUM_SUBCORES

TOTAL = BATCH * SEQ
PER_WORKER = TOTAL // NUM_WORKERS
CHUNK = 128
STEPS = PER_WORKER // CHUNK
NBUF = 4
SUPER = STEPS // NBUF


def _sc_gather(idx_hbm, table_hbm, out_hbm, idx_v, rows_v, gsems, wsems):
    wid = lax.axis_index("s") * NUM_CORES + lax.axis_index("c")
    base = wid * PER_WORKER

    def fire_gather(b, g):
        off = base + g * CHUNK
        pltpu.sync_copy(idx_hbm.at[pl.ds(off, CHUNK)], idx_v[b])
        pltpu.async_copy(table_hbm.at[idx_v[b]], rows_v[b], gsems[b])

    def wait_gather(b):
        pltpu.make_async_copy(table_hbm.at[idx_v[b]], rows_v[b], gsems[b]).wait()

    def fire_write(b, g):
        off = base + g * CHUNK
        pltpu.async_copy(rows_v[b], out_hbm.at[pl.ds(off, CHUNK)], wsems[b])

    def wait_write(b, g):
        off = base + g * CHUNK
        pltpu.make_async_copy(
            rows_v[b], out_hbm.at[pl.ds(off, CHUNK)], wsems[b]
        ).wait()

    for b in range(NBUF):
        fire_gather(b, b)

    def sstep(s, _):
        g0 = s * NBUF
        for b in range(NBUF):
            wait_gather(b)
            fire_write(b, g0 + b)
        for b in range(NBUF):
            h = g0 + NBUF + b

            @pl.when(h < STEPS)
            def _():
                wait_write(b, g0 + b)
                fire_gather(b, h)

        return ()

    lax.fori_loop(0, SUPER, sstep, ())

    for b in range(NBUF):
        wait_write(b, STEPS - NBUF + b)


@jax.jit
def kernel(prefix, table):
    idx = prefix.reshape(TOTAL).astype(jnp.int32)
    mesh = plsc.VectorSubcoreMesh(core_axis_name="c", subcore_axis_name="s")
    run = pl.kernel(
        _sc_gather,
        out_type=jax.ShapeDtypeStruct((TOTAL, HIDDEN), jnp.float32),
        mesh=mesh,
        scratch_types=[
            [pltpu.VMEM((CHUNK,), jnp.int32) for _ in range(NBUF)],
            [pltpu.VMEM((CHUNK, HIDDEN), jnp.float32) for _ in range(NBUF)],
            [pltpu.SemaphoreType.DMA for _ in range(NBUF)],
            [pltpu.SemaphoreType.DMA for _ in range(NBUF)],
        ],
    )
    out = run(idx, table)
    return out.reshape(BATCH, SEQ, HIDDEN)


# speedup vs baseline: 15.5584x; 4.5568x over previous
import jax
import jax.numpy as jnp
from jax import lax
from jax.experimental import pallas as pl
from jax.experimental.pallas import tpu as pltpu
from jax.experimental.pallas import tpu_sc as plsc

BATCH = 16384
SEQ = 200
HIDDEN = 128

NUM_CORES = 2
NUM_SUBCORES = 16
NUM_WORKERS = NUM_CORES * NUM_SUBCORES

PREFIX_ROWS = 200
TOTAL = BATCH * SEQ
PER_WORKER = TOTAL // NUM_WORKERS
CHUNK = 128
STEPS = PER_WORKER // CHUNK
NBUF = 4
SUPER = STEPS // NBUF


def _sc_gather(idx_hbm, table_hbm, out_hbm, table_sh, idx_v, rows_v, gsems, wsems):
    sid = lax.axis_index("s")
    wid = sid * NUM_CORES + lax.axis_index("c")
    base = wid * PER_WORKER

    @pl.when(sid == 0)
    def _():
        pltpu.sync_copy(table_hbm, table_sh)

    plsc.subcore_barrier()

    def fire_gather(b, g):
        off = base + g * CHUNK
        pltpu.sync_copy(idx_hbm.at[pl.ds(off, CHUNK)], idx_v[b])
        pltpu.async_copy(table_sh.at[idx_v[b]], rows_v[b], gsems[b])

    def wait_gather(b):
        pltpu.make_async_copy(table_sh.at[idx_v[b]], rows_v[b], gsems[b]).wait()

    def fire_write(b, g):
        off = base + g * CHUNK
        pltpu.async_copy(rows_v[b], out_hbm.at[pl.ds(off, CHUNK)], wsems[b])

    def wait_write(b, g):
        off = base + g * CHUNK
        pltpu.make_async_copy(
            rows_v[b], out_hbm.at[pl.ds(off, CHUNK)], wsems[b]
        ).wait()

    for b in range(NBUF):
        fire_gather(b, b)

    def sstep(s, _):
        g0 = s * NBUF
        for b in range(NBUF):
            wait_gather(b)
            fire_write(b, g0 + b)
        for b in range(NBUF):
            h = g0 + NBUF + b

            @pl.when(h < STEPS)
            def _():
                wait_write(b, g0 + b)
                fire_gather(b, h)

        return ()

    lax.fori_loop(0, SUPER, sstep, ())

    for b in range(NBUF):
        wait_write(b, STEPS - NBUF + b)


@jax.jit
def kernel(prefix, table):
    idx = prefix.reshape(TOTAL).astype(jnp.int32)
    mesh = plsc.VectorSubcoreMesh(core_axis_name="c", subcore_axis_name="s")
    run = pl.kernel(
        _sc_gather,
        out_type=jax.ShapeDtypeStruct((TOTAL, HIDDEN), jnp.float32),
        mesh=mesh,
        scratch_types=[
            pltpu.VMEM_SHARED((PREFIX_ROWS, HIDDEN), jnp.float32),
            [pltpu.VMEM((CHUNK,), jnp.int32) for _ in range(NBUF)],
            [pltpu.VMEM((CHUNK, HIDDEN), jnp.float32) for _ in range(NBUF)],
            [pltpu.SemaphoreType.DMA for _ in range(NBUF)],
            [pltpu.SemaphoreType.DMA for _ in range(NBUF)],
        ],
    )
    out = run(idx, table)
    return out.reshape(BATCH, SEQ, HIDDEN)
